# baseline (speedup 1.0000x reference)
import jax
import jax.numpy as jnp
from jax import lax
from jax.experimental import pallas as pl
from jax.experimental.pallas import tpu as pltpu
from jax.experimental.pallas import tpu_sc as plsc

NUM_ENT = 100000
NUM_REL = 1000
DIM = 50
BATCH = 16384
LMBDA = 0.1

NC = 2
NS = 16
L = 16
ROUNDS = 13
DPAD = 56
ROWH = NUM_ENT // 2
HLEN = 50176
HB1 = 49920
COV = 50048
JCH = 1024
NCHK = BATCH // JCH
OWN = BATCH // NS
RELROWS = DPAD * 32 + 32


def _prep_body(s_ref, x_ref, y_ref, z_ref, sn_ref, xn_ref, yn_ref, zn_ref,
               g_ref):
    s = s_ref[...]
    x = x_ref[...]
    y = y_ref[...]
    z = z_ref[...]
    sq = s * s + x * x + y * y + z * z
    denom = jnp.sqrt(sq)
    sn_ref[...] = s / denom
    xn_ref[...] = x / denom
    yn_ref[...] = y / denom
    zn_ref[...] = z / denom
    g_ref[...] = jnp.sum(sq, axis=1, keepdims=True)


def _prep(rs, rx, ry, rz):
    f = jax.ShapeDtypeStruct((NUM_REL, DIM), jnp.float32)
    g = jax.ShapeDtypeStruct((NUM_REL, 1), jnp.float32)
    return pl.pallas_call(_prep_body, out_shape=(f, f, f, f, g))(rs, rx, ry, rz)


def _pack_rel(pn, qn, un, vn):
    P = jnp.stack([pn, qn, un, vn])
    P = jnp.pad(P, ((0, 0), (0, 1024 - NUM_REL), (0, DPAD - DIM)))
    P = jnp.transpose(P, (2, 0, 1))
    P = P.reshape(DPAD * 4 * 8, 128)
    P = jnp.pad(P, ((0, RELROWS - DPAD * 32), (0, 0)))
    return P


def _sc_body(bh, bt, br, esT, exT, eyT, ezT, relblk, gblk,
             score_out, ss_out, gs_out,
             row_v, dbuf, idxh_ch, idxt_ch, outs, slabs, rel_v, g_v, idxr_v,
             score_v, vec_v, spm_val, spm_idx, spm_loc,
             sem_row, sem_o0, sem_o1, sem_idx):
    cid = lax.axis_index("c")
    sid = lax.axis_index("s")
    wid = cid * NS + sid
    tabs = (esT, exT, eyT, ezT)
    comp = lax.rem(sid, 4)
    half = lax.rem(lax.div(sid, 4), 2)
    dgrp = lax.div(sid, 8)
    osems = (sem_o0, sem_o1)

    z16 = jnp.zeros((L,), jnp.float32)

    @pl.when(sid == 0)
    def _():
        pltpu.sync_copy(bh, spm_idx.at[0])
        pltpu.sync_copy(bt, spm_idx.at[1])

    plsc.subcore_barrier()

    @pl.when(sid < 2)
    def _():
        lo_s = sid * COV
        hi_s = jnp.where(sid == 0, COV, NUM_ENT)
        hb_s = sid * HB1
        for ht in range(2):
            for ch in range(NCHK):
                pltpu.sync_copy(spm_idx.at[ht, pl.ds(ch * JCH, JCH)],
                                idxh_ch[0])

                @plsc.parallel_loop(0, JCH, step=L, unroll=2)
                def _(v):
                    sl = pl.ds(v, L)
                    raw = idxh_ch[0][sl]
                    idxt_ch[0][sl] = jnp.where(
                        (raw >= lo_s) & (raw < hi_s), raw - hb_s, HLEN)

                pltpu.sync_copy(idxt_ch[0],
                                spm_loc.at[sid, ht, pl.ds(ch * JCH, JCH)])

    pltpu.sync_copy(br.at[pl.ds(sid * OWN, OWN)], idxr_v)
    pltpu.sync_copy(gblk, g_v)
    for k in range(OWN // L):
        score_v[pl.ds(k * L, L)] = z16

    hb = half * HB1
    cov_lo = half * COV
    cov_hi = jnp.where(half == 0, COV, NUM_ENT)
    z16i = jnp.zeros((L,), jnp.int32)

    def stage_row(r):
        d = r * 4 + cid * 2 + dgrp
        dsafe = jnp.minimum(d, DIM - 1)
        dbuf[...] = z16i + dsafe
        for k in range(4):
            @pl.when(comp == k)
            def _():
                pltpu.async_copy(
                    tabs[k].at[dbuf.at[pl.ds(0, 1)], pl.ds(hb, HLEN)],
                    row_v.at[:, pl.ds(0, HLEN)], sem_row)

    def wait_row():
        pltpu.make_async_copy(
            tabs[0].at[dbuf.at[pl.ds(0, 1)], pl.ds(0, HLEN)],
            row_v.at[:, pl.ds(0, HLEN)], sem_row).wait()

    def drain_out(ob):
        for _ in range(2):
            pltpu.make_async_copy(esT.at[0, pl.ds(0, JCH)],
                                  outs[2 * ob], osems[ob]).wait()

    for k in range(128 // L):
        row_v[0, pl.ds(HLEN + k * L, L)] = z16
    stage_row(0)

    def gbody(v, acc):
        r16 = idxr_v[pl.ds(v * L, L)]
        gv = plsc.load_gather(g_v, [lax.shift_right_logical(r16, 7),
                                    jnp.bitwise_and(r16, 127)])
        return acc + gv

    gs_acc = lax.fori_loop(0, OWN // L, gbody, z16)
    gs_acc = jnp.where(cid == 0, gs_acc, z16)

    plsc.subcore_barrier()

    def round_body(r, ss_acc):
        hidx = [None, None]

        def fire_idx(ch):
            ib = ch % 2
            hidx[ib] = (
                pltpu.async_copy(spm_loc.at[half, 0, pl.ds(ch * JCH, JCH)],
                                 idxh_ch[ib], sem_idx),
                pltpu.async_copy(spm_loc.at[half, 1, pl.ds(ch * JCH, JCH)],
                                 idxt_ch[ib], sem_idx))

        fire_idx(0)
        wait_row()
        for ch in range(NCHK):
            ib = ch % 2
            for h in hidx[ib]:
                h.wait()
            if ch + 1 < NCHK:
                fire_idx(ch + 1)
            ob = ch % 2
            if ch >= 2:
                drain_out(ob)

            oh = outs[2 * ob]
            ot = outs[2 * ob + 1]
            ihc = idxh_ch[ib]
            itc = idxt_ch[ib]

            @plsc.parallel_loop(0, JCH, step=L, unroll=8)
            def _(v):
                sl = pl.ds(v, L)
                oh[sl] = plsc.load_gather(row_v, [z16i, ihc[sl]])
                ot[sl] = plsc.load_gather(row_v, [z16i, itc[sl]])
            pltpu.async_copy(
                outs[2 * ob],
                spm_val.at[dgrp, half * 8 + comp, pl.ds(ch * JCH, JCH)],
                osems[ob])
            pltpu.async_copy(
                outs[2 * ob + 1],
                spm_val.at[dgrp, half * 8 + 4 + comp, pl.ds(ch * JCH, JCH)],
                osems[ob])

        @pl.when(r < ROUNDS - 1)
        def _():
            stage_row(r + 1)

        drain_out(0)
        drain_out(1)
        plsc.subcore_barrier()

        for dloc in range(2):
            d = r * 4 + cid * 2 + dloc
            dmask = jnp.where(d < DIM, 1.0, 0.0).astype(jnp.float32)
            hrel = pltpu.async_copy(relblk.at[pl.ds(d * 32, 32)], rel_v,
                                    sem_row)
            for jp in range(2):
                jbase = jp * (OWN // 2)
                hs = [pltpu.async_copy(
                    spm_val.at[dloc, a, pl.ds(sid * OWN + jbase, OWN // 2)],
                    slabs[a], osems[jp]) for a in range(16)]
                if jp == 0:
                    hrel.wait()
                for h in hs:
                    h.wait()

                @plsc.parallel_loop(0, OWN // 2, step=L, unroll=4,
                                    carry=ss_acc)
                def cbody(v, acc):
                    sl = pl.ds(v, L)
                    r16 = idxr_v[pl.ds(jbase + v, L)]
                    rr = lax.shift_right_logical(r16, 7)
                    rc = jnp.bitwise_and(r16, 127)
                    p = plsc.load_gather(rel_v, [rr, rc])
                    q = plsc.load_gather(rel_v, [rr + 8, rc])
                    u = plsc.load_gather(rel_v, [rr + 16, rc])
                    w = plsc.load_gather(rel_v, [rr + 24, rc])
                    sa = slabs[0][sl] + slabs[8][sl]
                    xa = slabs[1][sl] + slabs[9][sl]
                    ya = slabs[2][sl] + slabs[10][sl]
                    za = slabs[3][sl] + slabs[11][sl]
                    sc = slabs[4][sl] + slabs[12][sl]
                    xc = slabs[5][sl] + slabs[13][sl]
                    yc = slabs[6][sl] + slabs[14][sl]
                    zc = slabs[7][sl] + slabs[15][sl]
                    A = sa * p - xa * q - ya * u - za * w
                    B = sa * q + p * xa + ya * w - u * za
                    C = sa * u + p * ya + za * q - w * xa
                    D = sa * w + p * za + xa * u - q * ya
                    score_v[pl.ds(jbase + v, L)] += (
                        A * sc + B * xc + C * yc + D * zc)
                    e2 = (sa * sa + xa * xa + ya * ya + za * za
                          + sc * sc + xc * xc + yc * yc + zc * zc)
                    return acc + dmask * e2

                ss_acc = cbody
        plsc.subcore_barrier()
        return ss_acc

    ss_acc = lax.fori_loop(0, ROUNDS, round_body, z16)

    pltpu.sync_copy(score_v, score_out.at[cid, pl.ds(sid * OWN, OWN)])
    vec_v[...] = ss_acc
    pltpu.sync_copy(vec_v, ss_out.at[wid])
    vec_v[...] = gs_acc
    pltpu.sync_copy(vec_v, gs_out.at[wid])


def _sc_main(bh, bt, br, esT, exT, eyT, ezT, relblk, gblk):
    mesh = plsc.VectorSubcoreMesh(core_axis_name="c", subcore_axis_name="s")
    scratch = [
        pltpu.VMEM((1, HLEN + 128), jnp.float32),
        pltpu.VMEM((L,), jnp.int32),
        [pltpu.VMEM((JCH,), jnp.int32) for _ in range(2)],
        [pltpu.VMEM((JCH,), jnp.int32) for _ in range(2)],
        [pltpu.VMEM((JCH,), jnp.float32) for _ in range(4)],
        [pltpu.VMEM((OWN // 2,), jnp.float32) for _ in range(16)],
        pltpu.VMEM((32, 128), jnp.float32),
        pltpu.VMEM((8, 128), jnp.float32),
        pltpu.VMEM((OWN,), jnp.int32),
        pltpu.VMEM((OWN,), jnp.float32),
        pltpu.VMEM((L,), jnp.float32),
        pltpu.VMEM_SHARED((2, 16, BATCH), jnp.float32),
        pltpu.VMEM_SHARED((2, BATCH), jnp.int32),
        pltpu.VMEM_SHARED((2, 2, BATCH), jnp.int32),
        pltpu.SemaphoreType.DMA,
        pltpu.SemaphoreType.DMA,
        pltpu.SemaphoreType.DMA,
        pltpu.SemaphoreType.DMA,
    ]
    out_type = (
        jax.ShapeDtypeStruct((NC, BATCH), jnp.float32),
        jax.ShapeDtypeStruct((NC * NS, L), jnp.float32),
        jax.ShapeDtypeStruct((NC * NS, L), jnp.float32),
    )
    run = pl.kernel(_sc_body, out_type=out_type, mesh=mesh,
                    compiler_params=pltpu.CompilerParams(
                        needs_layout_passes=False),
                    scratch_types=scratch)
    return run(bh, bt, br, esT, exT, eyT, ezT, relblk, gblk)


def _finish_body(sp_ref, tgt_ref, ss_ref, gs_ref, out_ref):
    score = -(sp_ref[0, :] + sp_ref[1, :])
    x = score * tgt_ref[...]
    sp = jnp.maximum(x, 0.0) + jnp.log(1.0 + jnp.exp(-jnp.abs(x)))
    loss = jnp.sum(sp) * (1.0 / BATCH)
    reg = (jnp.sum(ss_ref[...]) + jnp.sum(gs_ref[...])) * (1.0 / (BATCH * DIM))
    out_ref[...] = jnp.reshape(loss + LMBDA * reg, (1, 1))


def _finish(score_p, targets, ss, gs):
    out = pl.pallas_call(
        _finish_body,
        out_shape=jax.ShapeDtypeStruct((1, 1), jnp.float32),
    )(score_p, targets, ss, gs)
    return out[0, 0]


def kernel(batch_h, batch_r, batch_t, targets,
           emb_s_a, emb_x_a, emb_y_a, emb_z_a,
           rel_s_b, rel_x_b, rel_y_b, rel_z_b):
    pn, qn, un, vn, g = _prep(rel_s_b, rel_x_b, rel_y_b, rel_z_b)
    relblk = _pack_rel(pn, qn, un, vn)
    gblk = jnp.pad(g.reshape(1, NUM_REL), ((0, 0), (0, 24))).reshape(8, 128)
    score_p, ss, gs = _sc_main(
        batch_h, batch_t, batch_r,
        emb_s_a.T, emb_x_a.T, emb_y_a.T, emb_z_a.T, relblk, gblk)
    return _finish(score_p, targets, ss, gs)


# speedup vs baseline: 1.0817x; 1.0817x over previous
import jax
import jax.numpy as jnp
from jax import lax
from jax.experimental import pallas as pl
from jax.experimental.pallas import tpu as pltpu
from jax.experimental.pallas import tpu_sc as plsc

NUM_ENT = 100000
NUM_REL = 1000
DIM = 50
BATCH = 16384
LMBDA = 0.1

NC = 2
NS = 16
L = 16
ROUNDS = 13
DPAD = 56
ROWH = NUM_ENT // 2
HLEN = 50176
HB1 = 49920
COV = 50048
JCH = 2048
NCHK = BATCH // JCH
OWN = BATCH // NS
RELROWS = DPAD * 32 + 32


def _prep_body(s_ref, x_ref, y_ref, z_ref, sn_ref, xn_ref, yn_ref, zn_ref,
               g_ref):
    s = s_ref[...]
    x = x_ref[...]
    y = y_ref[...]
    z = z_ref[...]
    sq = s * s + x * x + y * y + z * z
    denom = jnp.sqrt(sq)
    sn_ref[...] = s / denom
    xn_ref[...] = x / denom
    yn_ref[...] = y / denom
    zn_ref[...] = z / denom
    g_ref[...] = jnp.sum(sq, axis=1, keepdims=True)


def _prep(rs, rx, ry, rz):
    f = jax.ShapeDtypeStruct((NUM_REL, DIM), jnp.float32)
    g = jax.ShapeDtypeStruct((NUM_REL, 1), jnp.float32)
    return pl.pallas_call(_prep_body, out_shape=(f, f, f, f, g))(rs, rx, ry, rz)


def _pack_rel(pn, qn, un, vn):
    P = jnp.stack([pn, qn, un, vn])
    P = jnp.pad(P, ((0, 0), (0, 1024 - NUM_REL), (0, DPAD - DIM)))
    P = jnp.transpose(P, (2, 0, 1))
    P = P.reshape(DPAD * 4 * 8, 128)
    P = jnp.pad(P, ((0, RELROWS - DPAD * 32), (0, 0)))
    return P


def _sc_body(bh, bt, br, esT, exT, eyT, ezT, relblk, gblk,
             score_out, ss_out, gs_out,
             row_v, dbuf, idxh_ch, idxt_ch, outs, slabs, rel_v, g_v, idxr_v,
             score_v, vec_v, spm_val, spm_idx,
             sem_row, sem_o0, sem_o1, sem_idx):
    cid = lax.axis_index("c")
    sid = lax.axis_index("s")
    wid = cid * NS + sid
    tabs = (esT, exT, eyT, ezT)
    comp = lax.rem(sid, 4)
    half = lax.rem(lax.div(sid, 4), 2)
    dgrp = lax.div(sid, 8)
    osems = (sem_o0, sem_o1)

    z16 = jnp.zeros((L,), jnp.float32)

    @pl.when(sid == 0)
    def _():
        pltpu.sync_copy(bh, spm_idx.at[0])
        pltpu.sync_copy(bt, spm_idx.at[1])

    pltpu.sync_copy(br.at[pl.ds(sid * OWN, OWN)], idxr_v)
    pltpu.sync_copy(gblk, g_v)
    for k in range(OWN // L):
        score_v[pl.ds(k * L, L)] = z16

    hb = half * HB1
    cov_lo = half * COV
    cov_hi = jnp.where(half == 0, COV, NUM_ENT)
    z16i = jnp.zeros((L,), jnp.int32)

    def stage_row(r):
        d = r * 4 + cid * 2 + dgrp
        dsafe = jnp.minimum(d, DIM - 1)
        dbuf[...] = z16i + dsafe
        for k in range(4):
            @pl.when(comp == k)
            def _():
                pltpu.async_copy(
                    tabs[k].at[dbuf.at[pl.ds(0, 1)], pl.ds(hb, HLEN)],
                    row_v.at[:, pl.ds(0, HLEN)], sem_row)

    def wait_row():
        pltpu.make_async_copy(
            tabs[0].at[dbuf.at[pl.ds(0, 1)], pl.ds(0, HLEN)],
            row_v.at[:, pl.ds(0, HLEN)], sem_row).wait()

    def drain_out(ob):
        for _ in range(2):
            pltpu.make_async_copy(esT.at[0, pl.ds(0, JCH)],
                                  outs[2 * ob], osems[ob]).wait()

    for k in range(128 // L):
        row_v[0, pl.ds(HLEN + k * L, L)] = z16
    stage_row(0)

    def gbody(v, acc):
        r16 = idxr_v[pl.ds(v * L, L)]
        gv = plsc.load_gather(g_v, [lax.shift_right_logical(r16, 7),
                                    jnp.bitwise_and(r16, 127)])
        return acc + gv

    gs_acc = lax.fori_loop(0, OWN // L, gbody, z16)
    gs_acc = jnp.where(cid == 0, gs_acc, z16)

    plsc.subcore_barrier()

    def round_body(r, ss_acc):
        hidx = [None, None]

        def fire_idx(ch):
            ib = ch % 2
            hidx[ib] = (
                pltpu.async_copy(spm_idx.at[0, pl.ds(ch * JCH, JCH)],
                                 idxh_ch[ib], sem_idx),
                pltpu.async_copy(spm_idx.at[1, pl.ds(ch * JCH, JCH)],
                                 idxt_ch[ib], sem_idx))

        fire_idx(0)
        wait_row()
        for ch in range(NCHK):
            ib = ch % 2
            for h in hidx[ib]:
                h.wait()
            if ch + 1 < NCHK:
                fire_idx(ch + 1)
            ob = ch % 2
            if ch >= 2:
                drain_out(ob)

            oh = outs[2 * ob]
            ot = outs[2 * ob + 1]
            ihc = idxh_ch[ib]
            itc = idxt_ch[ib]

            @plsc.parallel_loop(0, JCH, step=L, unroll=8)
            def _(v):
                sl = pl.ds(v, L)
                ih = ihc[sl]
                it = itc[sl]
                lh = jnp.where((ih >= cov_lo) & (ih < cov_hi),
                               ih - hb, HLEN)
                lt = jnp.where((it >= cov_lo) & (it < cov_hi),
                               it - hb, HLEN)
                oh[sl] = plsc.load_gather(row_v, [z16i, lh])
                ot[sl] = plsc.load_gather(row_v, [z16i, lt])
            pltpu.async_copy(
                outs[2 * ob],
                spm_val.at[dgrp, half * 8 + comp, pl.ds(ch * JCH, JCH)],
                osems[ob])
            pltpu.async_copy(
                outs[2 * ob + 1],
                spm_val.at[dgrp, half * 8 + 4 + comp, pl.ds(ch * JCH, JCH)],
                osems[ob])

        @pl.when(r < ROUNDS - 1)
        def _():
            stage_row(r + 1)

        drain_out(0)
        drain_out(1)
        plsc.subcore_barrier()

        for dloc in range(2):
            d = r * 4 + cid * 2 + dloc
            dmask = jnp.where(d < DIM, 1.0, 0.0).astype(jnp.float32)
            hrel = pltpu.async_copy(relblk.at[pl.ds(d * 32, 32)], rel_v,
                                    sem_row)
            for jp in range(2):
                jbase = jp * (OWN // 2)
                hs = [pltpu.async_copy(
                    spm_val.at[dloc, a, pl.ds(sid * OWN + jbase, OWN // 2)],
                    slabs[a], osems[jp]) for a in range(16)]
                if jp == 0:
                    hrel.wait()
                for h in hs:
                    h.wait()

                @plsc.parallel_loop(0, OWN // 2, step=L, unroll=4,
                                    carry=ss_acc)
                def cbody(v, acc):
                    sl = pl.ds(v, L)
                    r16 = idxr_v[pl.ds(jbase + v, L)]
                    rr = lax.shift_right_logical(r16, 7)
                    rc = jnp.bitwise_and(r16, 127)
                    p = plsc.load_gather(rel_v, [rr, rc])
                    q = plsc.load_gather(rel_v, [rr + 8, rc])
                    u = plsc.load_gather(rel_v, [rr + 16, rc])
                    w = plsc.load_gather(rel_v, [rr + 24, rc])
                    sa = slabs[0][sl] + slabs[8][sl]
                    xa = slabs[1][sl] + slabs[9][sl]
                    ya = slabs[2][sl] + slabs[10][sl]
                    za = slabs[3][sl] + slabs[11][sl]
                    sc = slabs[4][sl] + slabs[12][sl]
                    xc = slabs[5][sl] + slabs[13][sl]
                    yc = slabs[6][sl] + slabs[14][sl]
                    zc = slabs[7][sl] + slabs[15][sl]
                    A = sa * p - xa * q - ya * u - za * w
                    B = sa * q + p * xa + ya * w - u * za
                    C = sa * u + p * ya + za * q - w * xa
                    D = sa * w + p * za + xa * u - q * ya
                    score_v[pl.ds(jbase + v, L)] += (
                        A * sc + B * xc + C * yc + D * zc)
                    e2 = (sa * sa + xa * xa + ya * ya + za * za
                          + sc * sc + xc * xc + yc * yc + zc * zc)
                    return acc + dmask * e2

                ss_acc = cbody
        plsc.subcore_barrier()
        return ss_acc

    ss_acc = lax.fori_loop(0, ROUNDS, round_body, z16)

    pltpu.sync_copy(score_v, score_out.at[cid, pl.ds(sid * OWN, OWN)])
    vec_v[...] = ss_acc
    pltpu.sync_copy(vec_v, ss_out.at[wid])
    vec_v[...] = gs_acc
    pltpu.sync_copy(vec_v, gs_out.at[wid])


def _sc_main(bh, bt, br, esT, exT, eyT, ezT, relblk, gblk):
    mesh = plsc.VectorSubcoreMesh(core_axis_name="c", subcore_axis_name="s")
    scratch = [
        pltpu.VMEM((1, HLEN + 128), jnp.float32),
        pltpu.VMEM((L,), jnp.int32),
        [pltpu.VMEM((JCH,), jnp.int32) for _ in range(2)],
        [pltpu.VMEM((JCH,), jnp.int32) for _ in range(2)],
        [pltpu.VMEM((JCH,), jnp.float32) for _ in range(4)],
        [pltpu.VMEM((OWN // 2,), jnp.float32) for _ in range(16)],
        pltpu.VMEM((32, 128), jnp.float32),
        pltpu.VMEM((8, 128), jnp.float32),
        pltpu.VMEM((OWN,), jnp.int32),
        pltpu.VMEM((OWN,), jnp.float32),
        pltpu.VMEM((L,), jnp.float32),
        pltpu.VMEM_SHARED((2, 16, BATCH), jnp.float32),
        pltpu.VMEM_SHARED((2, BATCH), jnp.int32),
        pltpu.SemaphoreType.DMA,
        pltpu.SemaphoreType.DMA,
        pltpu.SemaphoreType.DMA,
        pltpu.SemaphoreType.DMA,
    ]
    out_type = (
        jax.ShapeDtypeStruct((NC, BATCH), jnp.float32),
        jax.ShapeDtypeStruct((NC * NS, L), jnp.float32),
        jax.ShapeDtypeStruct((NC * NS, L), jnp.float32),
    )
    run = pl.kernel(_sc_body, out_type=out_type, mesh=mesh,
                    compiler_params=pltpu.CompilerParams(
                        needs_layout_passes=False),
                    scratch_types=scratch)
    return run(bh, bt, br, esT, exT, eyT, ezT, relblk, gblk)


def _finish_body(sp_ref, tgt_ref, ss_ref, gs_ref, out_ref):
    score = -(sp_ref[0, :] + sp_ref[1, :])
    x = score * tgt_ref[...]
    sp = jnp.maximum(x, 0.0) + jnp.log(1.0 + jnp.exp(-jnp.abs(x)))
    loss = jnp.sum(sp) * (1.0 / BATCH)
    reg = (jnp.sum(ss_ref[...]) + jnp.sum(gs_ref[...])) * (1.0 / (BATCH * DIM))
    out_ref[...] = jnp.reshape(loss + LMBDA * reg, (1, 1))


def _finish(score_p, targets, ss, gs):
    out = pl.pallas_call(
        _finish_body,
        out_shape=jax.ShapeDtypeStruct((1, 1), jnp.float32),
    )(score_p, targets, ss, gs)
    return out[0, 0]


def kernel(batch_h, batch_r, batch_t, targets,
           emb_s_a, emb_x_a, emb_y_a, emb_z_a,
           rel_s_b, rel_x_b, rel_y_b, rel_z_b):
    pn, qn, un, vn, g = _prep(rel_s_b, rel_x_b, rel_y_b, rel_z_b)
    relblk = _pack_rel(pn, qn, un, vn)
    gblk = jnp.pad(g.reshape(1, NUM_REL), ((0, 0), (0, 24))).reshape(8, 128)
    score_p, ss, gs = _sc_main(
        batch_h, batch_t, batch_r,
        emb_s_a.T, emb_x_a.T, emb_y_a.T, emb_z_a.T, relblk, gblk)
    return _finish(score_p, targets, ss, gs)
